# baseline (speedup 1.0000x reference)
import jax
import jax.numpy as jnp
from jax import lax
from jax.experimental import pallas as pl
from jax.experimental.pallas import tpu as pltpu
from jax.experimental.pallas import tpu_sc as plsc

N_USERS = 100000
N_ITEMS = 1000
BATCH = 16384

_NC = 2
_NS = 16
_NW = _NC * _NS
_L = 16
_B_PER_W = BATCH // _NW
_NGRP = _B_PER_W // _L


def _gather_body(packed_hbm, tscores_hbm, out_hbm, pv, grp, vals, sem):
  wid = lax.axis_index("s") * _NC + lax.axis_index("c")
  base = wid * _B_PER_W
  pltpu.sync_copy(packed_hbm.at[pl.ds(base, _B_PER_W)], pv)

  def issue(g, _):
    p_vec = pv[pl.ds(g * _L, _L)]
    for j in range(_L):
      p = p_vec[j]
      it = p >> 20
      u0 = (p & 16383) * 8
      pltpu.async_copy(tscores_hbm.at[it, pl.ds(u0, 8)],
                       grp.at[pl.ds((g * _L + j) * 8, 8)], sem)
    return 0

  lax.fori_loop(0, _NGRP, issue, 0, unroll=2)

  pltpu.make_async_copy(out_hbm.at[pl.ds(0, _B_PER_W * 8)], grp, sem).wait()

  def extract(g, _):
    rows = lax.iota(jnp.int32, _L) * 8 + g * (_L * 8)
    lanes = (pv[pl.ds(g * _L, _L)] >> 17) & 7
    vals[pl.ds(g * _L, _L)] = plsc.load_gather(grp, [rows + lanes])
    return 0

  lax.fori_loop(0, _NGRP, extract, 0, unroll=2)

  pltpu.sync_copy(vals, out_hbm.at[pl.ds(base, _B_PER_W)])


def kernel(users, items, scores):
  users = users.astype(jnp.int32)
  items = items.astype(jnp.int32)
  packed = (items << 20) | ((users & 7) << 17) | (users >> 3)
  ts = scores.T
  mesh = plsc.VectorSubcoreMesh(core_axis_name="c", subcore_axis_name="s")
  return pl.kernel(
      _gather_body,
      out_type=jax.ShapeDtypeStruct((BATCH,), jnp.float32),
      mesh=mesh,
      scratch_types=[
          pltpu.VMEM((_B_PER_W,), jnp.int32),
          pltpu.VMEM((_B_PER_W * 8,), jnp.float32),
          pltpu.VMEM((_B_PER_W,), jnp.float32),
          pltpu.SemaphoreType.DMA,
      ],
      compiler_params=pltpu.CompilerParams(use_tc_tiling_on_sc=True,
                                           needs_layout_passes=False),
  )(packed, ts)


# speedup vs baseline: 1.0243x; 1.0243x over previous
import jax
import jax.numpy as jnp
from jax import lax
from jax.experimental import pallas as pl
from jax.experimental.pallas import tpu as pltpu
from jax.experimental.pallas import tpu_sc as plsc

N_USERS = 100000
N_ITEMS = 1000
BATCH = 16384

_NC = 2
_NS = 16
_NW = _NC * _NS
_L = 16
_B_PER_W = BATCH // _NW
_NGRP = _B_PER_W // _L


def _gather_body(packed_hbm, tscores_hbm, out_hbm, pv, grp, vals, sem):
  wid = lax.axis_index("s") * _NC + lax.axis_index("c")
  base = wid * _B_PER_W
  pltpu.sync_copy(packed_hbm.at[pl.ds(base, _B_PER_W)], pv)

  def issue(g, _):
    p_vec = pv[pl.ds(g * _L, _L)]
    for j in range(_L):
      p = p_vec[j]
      it = p >> 20
      u0 = (p & 16383) * 8
      pltpu.async_copy(tscores_hbm.at[it, pl.ds(u0, 8)],
                       grp.at[pl.ds((g * _L + j) * 8, 8)], sem)
    return 0

  lax.fori_loop(0, _NGRP, issue, 0)

  pltpu.make_async_copy(out_hbm.at[pl.ds(0, _B_PER_W * 8)], grp, sem).wait()

  def extract(g, _):
    rows = lax.iota(jnp.int32, _L) * 8 + g * (_L * 8)
    lanes = (pv[pl.ds(g * _L, _L)] >> 17) & 7
    vals[pl.ds(g * _L, _L)] = plsc.load_gather(grp, [rows + lanes])
    return 0

  lax.fori_loop(0, _NGRP, extract, 0)

  pltpu.sync_copy(vals, out_hbm.at[pl.ds(base, _B_PER_W)])


def kernel(users, items, scores):
  users = users.astype(jnp.int32)
  items = items.astype(jnp.int32)
  packed = (items << 20) | ((users & 7) << 17) | (users >> 3)
  ts = scores.T
  mesh = plsc.VectorSubcoreMesh(core_axis_name="c", subcore_axis_name="s")
  return pl.kernel(
      _gather_body,
      out_type=jax.ShapeDtypeStruct((BATCH,), jnp.float32),
      mesh=mesh,
      scratch_types=[
          pltpu.VMEM((_B_PER_W,), jnp.int32),
          pltpu.VMEM((_B_PER_W * 8,), jnp.float32),
          pltpu.VMEM((_B_PER_W,), jnp.float32),
          pltpu.SemaphoreType.DMA,
      ],
      compiler_params=pltpu.CompilerParams(use_tc_tiling_on_sc=True,
                                           needs_layout_passes=False),
  )(packed, ts)
